# baseline (speedup 1.0000x reference)
import functools

import jax
import jax.numpy as jnp
from jax import lax
from jax.experimental import pallas as pl
from jax.experimental.pallas import tpu as pltpu
from jax.experimental.pallas import tpu_sc as plsc

B, G, C, D = 16384, 26, 100, 32

H = 2
BH = B // H

BB = 256


_NEG = -3.4e38


def _argmax_body(x_ref, o_ref):
    lane = lax.broadcasted_iota(jnp.int32, (BB, 128), 1)
    nck = (G * C) // 128
    chunks = [x_ref[:, k * 128:(k + 1) * 128] for k in range(nck)]
    tail = x_ref[:, nck * 128:G * C]
    chunks.append(jnp.pad(tail, ((0, 0), (0, 128 - tail.shape[1])),
                          constant_values=_NEG))
    cols = []
    for g in range(G):
        col0 = g * C
        k, s = col0 // 128, col0 % 128
        rk = jnp.roll(chunks[k], -s, axis=1) if s else chunks[k]
        if s + C <= 128:
            merged = jnp.where(lane < C, rk, _NEG)
        else:
            rk1 = jnp.roll(chunks[k + 1], -s, axis=1)
            merged = jnp.where(lane < 128 - s, rk,
                               jnp.where(lane < C, rk1, _NEG))
        idx = jnp.argmax(merged, axis=-1)
        cols.append(idx.astype(jnp.int32) + (128 * k + s))
    o_ref[...] = jnp.stack(cols, axis=1)


def _tc_argmax(x, h):
    return pl.pallas_call(
        _argmax_body,
        grid=(BH // BB,),
        in_specs=[pl.BlockSpec((BB, G * C), lambda i, h=h: (h * (BH // BB) + i, 0))],
        out_specs=pl.BlockSpec((BB, G), lambda i: (i, 0)),
        out_shape=jax.ShapeDtypeStruct((BH, G), jnp.int32),
    )(x)


NC, NS = 2, 16
NW = NC * NS
TOKH = BH * G
ICH = 64
NCHUNK = TOKH // ICH
CPW = NCHUNK // NW
KIN = 8
NOUT = CPW // KIN


@functools.lru_cache(maxsize=1)
def _build_sc_gather():
    @functools.partial(
        pl.kernel,
        mesh=plsc.VectorSubcoreMesh(core_axis_name="c", subcore_axis_name="s"),
        out_type=jax.ShapeDtypeStruct((NCHUNK, ICH, D), jnp.float32),
        scratch_types=[
            pltpu.VMEM((CPW, ICH), jnp.int32),
            pltpu.VMEM((KIN, ICH, D), jnp.float32),
            pltpu.SemaphoreType.DMA,
        ],
        compiler_params=pltpu.CompilerParams(use_tc_tiling_on_sc=False),
    )
    def _sc_gather(fidx_hbm, table_hbm, out_hbm, idx_v, emb_v, sem):
        wid = lax.axis_index("s") * NC + lax.axis_index("c")
        base = wid * CPW
        pltpu.sync_copy(fidx_hbm.at[pl.ds(base, CPW)], idx_v)

        def step(t, carry):
            copies = [
                pltpu.async_copy(
                    table_hbm.at[idx_v.at[t * KIN + j]], emb_v.at[j], sem)
                for j in range(KIN)
            ]
            for cp in copies:
                cp.wait()
            pltpu.sync_copy(emb_v, out_hbm.at[pl.ds(base + t * KIN, KIN)])
            return carry

        lax.fori_loop(0, NOUT, step, 0)

    return _sc_gather


def kernel(x, tables):
    table2 = tables.reshape(G * C, D)
    sc = _build_sc_gather()
    halves = []
    for h in range(H):
        fidx = _tc_argmax(x, h)
        fidx2 = fidx.reshape(NCHUNK, ICH)
        out3 = sc(fidx2, table2)
        halves.append(out3)
    return jnp.concatenate(halves, axis=0).reshape(B, G * D)


# speedup vs baseline: 1.5624x; 1.5624x over previous
import functools

import jax
import jax.numpy as jnp
from jax import lax
from jax.experimental import pallas as pl
from jax.experimental.pallas import tpu as pltpu
from jax.experimental.pallas import tpu_sc as plsc

B, G, C, D = 16384, 26, 100, 32

H = 2
BH = B // H

BB = 256


_NEG = -3.4e38


def _argmax_body(x_ref, o_ref):
    lane = lax.broadcasted_iota(jnp.int32, (BB, 128), 1)
    nck = (G * C) // 128
    chunks = [x_ref[:, k * 128:(k + 1) * 128] for k in range(nck)]
    tail = x_ref[:, nck * 128:G * C]
    chunks.append(jnp.pad(tail, ((0, 0), (0, 128 - tail.shape[1])),
                          constant_values=_NEG))
    cols = []
    for g in range(G):
        col0 = g * C
        k, s = col0 // 128, col0 % 128
        rk = jnp.roll(chunks[k], -s, axis=1) if s else chunks[k]
        if s + C <= 128:
            merged = jnp.where(lane < C, rk, _NEG)
        else:
            rk1 = jnp.roll(chunks[k + 1], -s, axis=1)
            merged = jnp.where(lane < 128 - s, rk,
                               jnp.where(lane < C, rk1, _NEG))
        idx = jnp.argmax(merged, axis=-1)
        cols.append(idx.astype(jnp.int32) + (128 * k + s))
    o_ref[...] = jnp.stack(cols, axis=1)


def _tc_argmax(x, h):
    return pl.pallas_call(
        _argmax_body,
        grid=(BH // BB,),
        in_specs=[pl.BlockSpec((BB, G * C), lambda i, h=h: (h * (BH // BB) + i, 0))],
        out_specs=pl.BlockSpec((BB, G), lambda i: (i, 0)),
        out_shape=jax.ShapeDtypeStruct((BH, G), jnp.int32),
    )(x)


NC, NS = 2, 16
NW = NC * NS
TOKH = BH * G
ICH = 64
NCHUNK = TOKH // ICH
CPW = NCHUNK // NW
KIN = 8
NOUT = CPW // KIN


@functools.lru_cache(maxsize=1)
def _build_sc_gather():
    @functools.partial(
        pl.kernel,
        mesh=plsc.VectorSubcoreMesh(core_axis_name="c", subcore_axis_name="s"),
        out_type=jax.ShapeDtypeStruct((NCHUNK, ICH, D), jnp.float32),
        scratch_types=[
            pltpu.VMEM((CPW, ICH), jnp.int32),
            pltpu.VMEM((KIN, ICH, D), jnp.float32),
            pltpu.SemaphoreType.DMA,
        ],
        compiler_params=pltpu.CompilerParams(use_tc_tiling_on_sc=False),
    )
    def _sc_gather(fidx_hbm, table_hbm, out_hbm, idx_v, emb_v, sem):
        wid = lax.axis_index("s") * NC + lax.axis_index("c")
        base = wid * CPW
        pltpu.sync_copy(fidx_hbm.at[pl.ds(base, CPW)], idx_v)

        def step(t, carry):
            copies = [
                pltpu.async_copy(
                    table_hbm.at[idx_v.at[t * KIN + j]], emb_v.at[j], sem)
                for j in range(KIN)
            ]
            for cp in copies:
                cp.wait()
            pltpu.sync_copy(emb_v, out_hbm.at[pl.ds(base + t * KIN, KIN)])
            return carry

        lax.fori_loop(0, NOUT, step, 0)

    return _sc_gather


def kernel(x, tables):
    table2 = tables.reshape(G * C, D)
    sc = _build_sc_gather()
    halves = []
    for h in range(H):
        fidx = _tc_argmax(x, h)
        fidx2 = fidx.reshape(NCHUNK, ICH)
        out3 = sc(fidx2, table2)
        halves.append(out3.reshape(BH, G * D))
    return jnp.stack(halves, axis=0).reshape(B, G * D)
